# Initial kernel scaffold; baseline (speedup 1.0000x reference)
import jax
import jax.numpy as jnp
from jax.experimental import pallas as pl


def kernel(x, edge_index, W1, b1, W2, b2):
    raise NotImplementedError("write your pallas kernel here")


# speedup vs baseline: 12.5996x; 12.5996x over previous
import functools

import jax
import jax.numpy as jnp
from jax import lax
from jax.experimental import pallas as pl
from jax.experimental.pallas import tpu as pltpu
from jax.experimental.pallas import tpu_sc as plsc

N = 10000
E = 320000
D_IN = 128
NHID = 64
NLAT = 16

NC = 2
NS = 16
L = 16
NW = NC * NS
CHUNK = 128
CH_PER_W = 80
EPW = CH_PER_W * CHUNK
EP = NW * EPW
NP = 10112
ROWS_PER_TILE = NP // NS

_MESH = dict(core_axis_name="c", subcore_axis_name="s")


def _deg_body(dst_hbm, out_hbm, idx_v, deg_v):
  cid = lax.axis_index("c")
  sid = lax.axis_index("s")
  w = cid * NS + sid
  pltpu.sync_copy(dst_hbm.at[pl.ds(w * EPW, EPW)], idx_v)

  @pl.loop(0, NP // L)
  def _(i):
    deg_v[pl.ds(i * L, L)] = jnp.zeros((L,), jnp.float32)

  ones = jnp.ones((L,), jnp.float32)

  @pl.loop(0, EPW // L)
  def _(j):
    idx = idx_v[pl.ds(j * L, L)]
    plsc.addupdate_scatter(deg_v, [idx], ones)

  pltpu.sync_copy(deg_v, out_hbm.at[w])


def _deg(dstp):
  mesh = plsc.VectorSubcoreMesh(**_MESH)
  return pl.kernel(
      _deg_body,
      out_type=jax.ShapeDtypeStruct((NW, NP), jnp.float32),
      mesh=mesh,
      scratch_types=[
          pltpu.VMEM((EPW,), jnp.int32),
          pltpu.VMEM((NP,), jnp.float32),
      ],
      compiler_params=pltpu.CompilerParams(needs_layout_passes=False),
  )(dstp)


def _agg_body(hp_hbm, srcm_hbm, dstm_hbm, zeros_hbm, out_hbm,
              src2_v, dst2_v, rows_v, acc_sh):
  cid = lax.axis_index("c")
  sid = lax.axis_index("s")
  w = cid * NS + sid
  r0 = sid * ROWS_PER_TILE
  pltpu.sync_copy(zeros_hbm.at[pl.ds(r0, ROWS_PER_TILE)],
                  acc_sh.at[pl.ds(r0, ROWS_PER_TILE)])
  pltpu.sync_copy(srcm_hbm.at[pl.ds(w * CH_PER_W, CH_PER_W)], src2_v)
  pltpu.sync_copy(dstm_hbm.at[pl.ds(w * CH_PER_W, CH_PER_W)], dst2_v)
  plsc.subcore_barrier()

  @pl.loop(0, CH_PER_W)
  def _(k):
    pltpu.sync_copy(hp_hbm.at[src2_v.at[k]], rows_v)
    pltpu.sync_copy(rows_v, acc_sh.at[dst2_v.at[k]], add=True)

  plsc.subcore_barrier()
  pltpu.sync_copy(acc_sh.at[pl.ds(r0, ROWS_PER_TILE)],
                  out_hbm.at[cid, pl.ds(r0, ROWS_PER_TILE)])


def _agg(hp, srcm, dstm, zeros_np, f):
  mesh = plsc.VectorSubcoreMesh(**_MESH)
  return pl.kernel(
      _agg_body,
      out_type=jax.ShapeDtypeStruct((NC, NP, f), jnp.float32),
      mesh=mesh,
      scratch_types=[
          pltpu.VMEM((CH_PER_W, CHUNK), jnp.int32),
          pltpu.VMEM((CH_PER_W, CHUNK), jnp.int32),
          pltpu.VMEM((CHUNK, f), jnp.float32),
          pltpu.VMEM_SHARED((NP, f), jnp.float32),
      ],
      compiler_params=pltpu.CompilerParams(use_tc_tiling_on_sc=False),
  )(hp, srcm, dstm, zeros_np)


def _mm1_body(x_ref, w1_ref, h1_ref):
  h1_ref[...] = jnp.dot(x_ref[...], w1_ref[...],
                        preferred_element_type=jnp.float32,
                        precision=lax.Precision.HIGHEST)


def _mm1(x, W1):
  bm = 1000
  return pl.pallas_call(
      _mm1_body,
      grid=(N // bm,),
      in_specs=[
          pl.BlockSpec((bm, D_IN), lambda i: (i, 0)),
          pl.BlockSpec((D_IN, NHID), lambda i: (0, 0)),
      ],
      out_specs=pl.BlockSpec((bm, NHID), lambda i: (i, 0)),
      out_shape=jax.ShapeDtypeStruct((N, NHID), jnp.float32),
  )(x, W1)


def _dinv_body(degp_ref, h1_ref, dinv_ref, h1p_ref):
  deg = jnp.sum(degp_ref[...], axis=0)[:N] + 1.0
  dinv = lax.rsqrt(deg)[:, None]
  dinv_ref[...] = dinv
  h1p_ref[...] = h1_ref[...] * dinv


def _dinv(degp, h1):
  return pl.pallas_call(
      _dinv_body,
      grid=(1,),
      in_specs=[
          pl.BlockSpec((NW, NP), lambda i: (0, 0)),
          pl.BlockSpec((N, NHID), lambda i: (0, 0)),
      ],
      out_specs=[
          pl.BlockSpec((N, 1), lambda i: (0, 0)),
          pl.BlockSpec((N, NHID), lambda i: (0, 0)),
      ],
      out_shape=[
          jax.ShapeDtypeStruct((N, 1), jnp.float32),
          jax.ShapeDtypeStruct((N, NHID), jnp.float32),
      ],
  )(degp, h1)


def _mid_body(p_ref, h1p_ref, dinv_ref, b1_ref, w2_ref, h2p_ref):
  dinv = dinv_ref[...]
  s = (p_ref[0] + p_ref[1] + h1p_ref[...]) * dinv + b1_ref[...]
  h = jnp.maximum(s, 0.0)
  h2 = jnp.dot(h, w2_ref[...], preferred_element_type=jnp.float32,
               precision=lax.Precision.HIGHEST)
  h2p_ref[...] = h2 * dinv


def _mid(p, h1p, dinv, b1, W2):
  bm = 1000
  return pl.pallas_call(
      _mid_body,
      grid=(N // bm,),
      in_specs=[
          pl.BlockSpec((NC, bm, NHID), lambda i: (0, i, 0)),
          pl.BlockSpec((bm, NHID), lambda i: (i, 0)),
          pl.BlockSpec((bm, 1), lambda i: (i, 0)),
          pl.BlockSpec((1, NHID), lambda i: (0, 0)),
          pl.BlockSpec((NHID, NLAT), lambda i: (0, 0)),
      ],
      out_specs=pl.BlockSpec((bm, NLAT), lambda i: (i, 0)),
      out_shape=jax.ShapeDtypeStruct((N, NLAT), jnp.float32),
  )(p, h1p, dinv, b1, W2)


def _zk_body(q_ref, h2p_ref, dinv_ref, b2_ref, z_ref):
  z_ref[...] = ((q_ref[0] + q_ref[1] + h2p_ref[...]) * dinv_ref[...]
                + b2_ref[...])


def _zk(q, h2p, dinv, b2):
  bm = 1000
  return pl.pallas_call(
      _zk_body,
      grid=(N // bm,),
      in_specs=[
          pl.BlockSpec((NC, bm, NLAT), lambda i: (0, i, 0)),
          pl.BlockSpec((bm, NLAT), lambda i: (i, 0)),
          pl.BlockSpec((bm, 1), lambda i: (i, 0)),
          pl.BlockSpec((1, NLAT), lambda i: (0, 0)),
      ],
      out_specs=pl.BlockSpec((bm, NLAT), lambda i: (i, 0)),
      out_shape=jax.ShapeDtypeStruct((N, NLAT), jnp.float32),
  )(q, h2p, dinv, b2)


def _dec_body(zi_ref, zj_ref, out_ref):
  out_ref[...] = lax.dot_general(
      zi_ref[...], zj_ref[...],
      dimension_numbers=(((1,), (1,)), ((), ())),
      preferred_element_type=jnp.float32,
      precision=lax.Precision.HIGHEST)


def _dec(z):
  bm = 400
  return pl.pallas_call(
      _dec_body,
      grid=(N // bm,),
      in_specs=[
          pl.BlockSpec((bm, NLAT), lambda i: (i, 0)),
          pl.BlockSpec((N, NLAT), lambda i: (0, 0)),
      ],
      out_specs=pl.BlockSpec((bm, N), lambda i: (i, 0)),
      out_shape=jax.ShapeDtypeStruct((N, N), jnp.float32),
      compiler_params=pltpu.CompilerParams(
          dimension_semantics=("parallel",)),
  )(z, z)


def kernel(x, edge_index, W1, b1, W2, b2):
  src = edge_index[0]
  dst = edge_index[1]
  pad = EP - E
  srcp = jnp.concatenate([src, jnp.zeros((pad,), jnp.int32)])
  dstp = jnp.concatenate([dst, jnp.full((pad,), N, jnp.int32)])
  srcm = srcp.reshape(NW * CH_PER_W, CHUNK)
  dstm = dstp.reshape(NW * CH_PER_W, CHUNK)
  zeros64 = jnp.zeros((NP, NHID), jnp.float32)
  zeros16 = jnp.zeros((NP, NLAT), jnp.float32)

  degp = _deg(dstp)
  h1 = _mm1(x, W1)
  dinv, h1p = _dinv(degp, h1)
  p = _agg(h1p, srcm, dstm, zeros64, NHID)
  h2p = _mid(p[:, :N], h1p, dinv, b1.reshape(1, NHID), W2)
  q = _agg(h2p, srcm, dstm, zeros16, NLAT)
  z = _zk(q[:, :N], h2p, dinv, b2.reshape(1, NLAT))
  return _dec(z)


# speedup vs baseline: 13.1571x; 1.0443x over previous
import functools

import jax
import jax.numpy as jnp
from jax import lax
from jax.experimental import pallas as pl
from jax.experimental.pallas import tpu as pltpu
from jax.experimental.pallas import tpu_sc as plsc

N = 10000
E = 320000
D_IN = 128
NHID = 64
NLAT = 16

NC = 2
NS = 16
L = 16
NW = NC * NS
CHUNK = 128
CH_PER_W = 80
EPW = CH_PER_W * CHUNK
EP = NW * EPW
NP = 10112
ROWS_PER_TILE = NP // NS

_MESH = dict(core_axis_name="c", subcore_axis_name="s")


def _deg_body(dst_hbm, out_hbm, idx_v, deg_v):
  cid = lax.axis_index("c")
  sid = lax.axis_index("s")
  w = cid * NS + sid
  pltpu.sync_copy(dst_hbm.at[pl.ds(w * EPW, EPW)], idx_v)

  @pl.loop(0, NP // L)
  def _(i):
    deg_v[pl.ds(i * L, L)] = jnp.zeros((L,), jnp.float32)

  ones = jnp.ones((L,), jnp.float32)

  @pl.loop(0, EPW // L)
  def _(j):
    idx = idx_v[pl.ds(j * L, L)]
    plsc.addupdate_scatter(deg_v, [idx], ones)

  pltpu.sync_copy(deg_v, out_hbm.at[w])


def _deg(dstp):
  mesh = plsc.VectorSubcoreMesh(**_MESH)
  return pl.kernel(
      _deg_body,
      out_type=jax.ShapeDtypeStruct((NW, NP), jnp.float32),
      mesh=mesh,
      scratch_types=[
          pltpu.VMEM((EPW,), jnp.int32),
          pltpu.VMEM((NP,), jnp.float32),
      ],
      compiler_params=pltpu.CompilerParams(needs_layout_passes=False),
  )(dstp)


def _agg_body(hp_hbm, srcm_hbm, dstm_hbm, zeros_hbm, out_hbm,
              src2_v, dst2_v, rows0_v, rows1_v, acc_sh,
              sg0, sg1, ss0, ss1):
  cid = lax.axis_index("c")
  sid = lax.axis_index("s")
  w = cid * NS + sid
  r0 = sid * ROWS_PER_TILE
  pltpu.sync_copy(zeros_hbm.at[pl.ds(r0, ROWS_PER_TILE)],
                  acc_sh.at[pl.ds(r0, ROWS_PER_TILE)])
  pltpu.sync_copy(srcm_hbm.at[pl.ds(w * CH_PER_W, CH_PER_W)], src2_v)
  pltpu.sync_copy(dstm_hbm.at[pl.ds(w * CH_PER_W, CH_PER_W)], dst2_v)
  plsc.subcore_barrier()

  pltpu.async_copy(hp_hbm.at[src2_v.at[0]], rows0_v, sg0)

  @pl.loop(0, CH_PER_W, step=2)
  def _(k):
    pltpu.make_async_copy(hp_hbm.at[src2_v.at[k]], rows0_v, sg0).wait()

    @pl.when(k > 0)
    def _():
      pltpu.make_async_copy(rows1_v, acc_sh.at[dst2_v.at[k - 1]], ss1).wait()

    pltpu.async_copy(hp_hbm.at[src2_v.at[k + 1]], rows1_v, sg1)
    pltpu.async_copy(rows0_v, acc_sh.at[dst2_v.at[k]], ss0, add=True)
    pltpu.make_async_copy(hp_hbm.at[src2_v.at[k + 1]], rows1_v, sg1).wait()
    pltpu.make_async_copy(rows0_v, acc_sh.at[dst2_v.at[k]], ss0).wait()
    knext = jnp.where(k + 2 >= CH_PER_W, 0, k + 2)
    pltpu.async_copy(hp_hbm.at[src2_v.at[knext]], rows0_v, sg0)
    pltpu.async_copy(rows1_v, acc_sh.at[dst2_v.at[k + 1]], ss1, add=True)

  pltpu.make_async_copy(rows1_v, acc_sh.at[dst2_v.at[CH_PER_W - 1]],
                        ss1).wait()
  pltpu.make_async_copy(hp_hbm.at[src2_v.at[0]], rows0_v, sg0).wait()

  plsc.subcore_barrier()
  pltpu.sync_copy(acc_sh.at[pl.ds(r0, ROWS_PER_TILE)],
                  out_hbm.at[cid, pl.ds(r0, ROWS_PER_TILE)])


def _agg(hp, srcm, dstm, zeros_np, f):
  mesh = plsc.VectorSubcoreMesh(**_MESH)
  return pl.kernel(
      _agg_body,
      out_type=jax.ShapeDtypeStruct((NC, NP, f), jnp.float32),
      mesh=mesh,
      scratch_types=[
          pltpu.VMEM((CH_PER_W, CHUNK), jnp.int32),
          pltpu.VMEM((CH_PER_W, CHUNK), jnp.int32),
          pltpu.VMEM((CHUNK, f), jnp.float32),
          pltpu.VMEM((CHUNK, f), jnp.float32),
          pltpu.VMEM_SHARED((NP, f), jnp.float32),
          pltpu.SemaphoreType.DMA,
          pltpu.SemaphoreType.DMA,
          pltpu.SemaphoreType.DMA,
          pltpu.SemaphoreType.DMA,
      ],
      compiler_params=pltpu.CompilerParams(use_tc_tiling_on_sc=False),
  )(hp, srcm, dstm, zeros_np)


def _mm1_body(x_ref, w1_ref, h1_ref):
  h1_ref[...] = jnp.dot(x_ref[...], w1_ref[...],
                        preferred_element_type=jnp.float32,
                        precision=lax.Precision.HIGHEST)


def _mm1(x, W1):
  bm = 1000
  return pl.pallas_call(
      _mm1_body,
      grid=(N // bm,),
      in_specs=[
          pl.BlockSpec((bm, D_IN), lambda i: (i, 0)),
          pl.BlockSpec((D_IN, NHID), lambda i: (0, 0)),
      ],
      out_specs=pl.BlockSpec((bm, NHID), lambda i: (i, 0)),
      out_shape=jax.ShapeDtypeStruct((N, NHID), jnp.float32),
  )(x, W1)


def _dinv_body(degp_ref, h1_ref, dinv_ref, h1p_ref):
  deg = jnp.sum(degp_ref[...], axis=0)[:N] + 1.0
  dinv = lax.rsqrt(deg)[:, None]
  dinv_ref[...] = dinv
  h1p_ref[...] = h1_ref[...] * dinv


def _dinv(degp, h1):
  return pl.pallas_call(
      _dinv_body,
      grid=(1,),
      in_specs=[
          pl.BlockSpec((NW, NP), lambda i: (0, 0)),
          pl.BlockSpec((N, NHID), lambda i: (0, 0)),
      ],
      out_specs=[
          pl.BlockSpec((N, 1), lambda i: (0, 0)),
          pl.BlockSpec((N, NHID), lambda i: (0, 0)),
      ],
      out_shape=[
          jax.ShapeDtypeStruct((N, 1), jnp.float32),
          jax.ShapeDtypeStruct((N, NHID), jnp.float32),
      ],
  )(degp, h1)


def _mid_body(p_ref, h1p_ref, dinv_ref, b1_ref, w2_ref, h2p_ref):
  dinv = dinv_ref[...]
  s = (p_ref[0] + p_ref[1] + h1p_ref[...]) * dinv + b1_ref[...]
  h = jnp.maximum(s, 0.0)
  h2 = jnp.dot(h, w2_ref[...], preferred_element_type=jnp.float32,
               precision=lax.Precision.HIGHEST)
  h2p_ref[...] = h2 * dinv


def _mid(p, h1p, dinv, b1, W2):
  bm = 1000
  return pl.pallas_call(
      _mid_body,
      grid=(N // bm,),
      in_specs=[
          pl.BlockSpec((NC, bm, NHID), lambda i: (0, i, 0)),
          pl.BlockSpec((bm, NHID), lambda i: (i, 0)),
          pl.BlockSpec((bm, 1), lambda i: (i, 0)),
          pl.BlockSpec((1, NHID), lambda i: (0, 0)),
          pl.BlockSpec((NHID, NLAT), lambda i: (0, 0)),
      ],
      out_specs=pl.BlockSpec((bm, NLAT), lambda i: (i, 0)),
      out_shape=jax.ShapeDtypeStruct((N, NLAT), jnp.float32),
  )(p, h1p, dinv, b1, W2)


def _zk_body(q_ref, h2p_ref, dinv_ref, b2_ref, z_ref):
  z_ref[...] = ((q_ref[0] + q_ref[1] + h2p_ref[...]) * dinv_ref[...]
                + b2_ref[...])


def _zk(q, h2p, dinv, b2):
  bm = 1000
  return pl.pallas_call(
      _zk_body,
      grid=(N // bm,),
      in_specs=[
          pl.BlockSpec((NC, bm, NLAT), lambda i: (0, i, 0)),
          pl.BlockSpec((bm, NLAT), lambda i: (i, 0)),
          pl.BlockSpec((bm, 1), lambda i: (i, 0)),
          pl.BlockSpec((1, NLAT), lambda i: (0, 0)),
      ],
      out_specs=pl.BlockSpec((bm, NLAT), lambda i: (i, 0)),
      out_shape=jax.ShapeDtypeStruct((N, NLAT), jnp.float32),
  )(q, h2p, dinv, b2)


def _dec_body(zi_ref, zj_ref, out_ref):
  out_ref[...] = lax.dot_general(
      zi_ref[...], zj_ref[...],
      dimension_numbers=(((1,), (1,)), ((), ())),
      preferred_element_type=jnp.float32,
      precision=lax.Precision.HIGHEST)


def _dec(z):
  bm = 400
  return pl.pallas_call(
      _dec_body,
      grid=(N // bm,),
      in_specs=[
          pl.BlockSpec((bm, NLAT), lambda i: (i, 0)),
          pl.BlockSpec((N, NLAT), lambda i: (0, 0)),
      ],
      out_specs=pl.BlockSpec((bm, N), lambda i: (i, 0)),
      out_shape=jax.ShapeDtypeStruct((N, N), jnp.float32),
      compiler_params=pltpu.CompilerParams(
          dimension_semantics=("parallel",)),
  )(z, z)


def kernel(x, edge_index, W1, b1, W2, b2):
  src = edge_index[0]
  dst = edge_index[1]
  pad = EP - E
  srcp = jnp.concatenate([src, jnp.zeros((pad,), jnp.int32)])
  dstp = jnp.concatenate([dst, jnp.full((pad,), N, jnp.int32)])
  srcm = srcp.reshape(NW * CH_PER_W, CHUNK)
  dstm = dstp.reshape(NW * CH_PER_W, CHUNK)
  zeros64 = jnp.zeros((NP, NHID), jnp.float32)
  zeros16 = jnp.zeros((NP, NLAT), jnp.float32)

  degp = _deg(dstp)
  h1 = _mm1(x, W1)
  dinv, h1p = _dinv(degp, h1)
  p = _agg(h1p, srcm, dstm, zeros64, NHID)
  h2p = _mid(p[:, :N], h1p, dinv, b1.reshape(1, NHID), W2)
  q = _agg(h2p, srcm, dstm, zeros16, NLAT)
  z = _zk(q[:, :N], h2p, dinv, b2.reshape(1, NLAT))
  return _dec(z)


# speedup vs baseline: 18.2018x; 1.3834x over previous
import functools

import jax
import jax.numpy as jnp
from jax import lax
from jax.experimental import pallas as pl
from jax.experimental.pallas import tpu as pltpu
from jax.experimental.pallas import tpu_sc as plsc

N = 10000
E = 320000
D_IN = 128
NHID = 64
NLAT = 16

NC = 2
NS = 16
L = 16
NW = NC * NS
CHUNK = 128
CH_PER_W = 80
EPW = CH_PER_W * CHUNK
EP = NW * EPW
NP = 10112
ROWS_PER_TILE = NP // NS

_MESH = dict(core_axis_name="c", subcore_axis_name="s")


def _deg_body(dst_hbm, out_hbm, idx_v, deg_v):
  cid = lax.axis_index("c")
  sid = lax.axis_index("s")
  w = cid * NS + sid
  pltpu.sync_copy(dst_hbm.at[pl.ds(w * EPW, EPW)], idx_v)

  @pl.loop(0, NP // L)
  def _(i):
    deg_v[pl.ds(i * L, L)] = jnp.zeros((L,), jnp.float32)

  ones = jnp.ones((L,), jnp.float32)

  @pl.loop(0, EPW // L)
  def _(j):
    idx = idx_v[pl.ds(j * L, L)]
    plsc.addupdate_scatter(deg_v, [idx], ones)

  pltpu.sync_copy(deg_v, out_hbm.at[w])


def _deg(dstp):
  mesh = plsc.VectorSubcoreMesh(**_MESH)
  return pl.kernel(
      _deg_body,
      out_type=jax.ShapeDtypeStruct((NW, NP), jnp.float32),
      mesh=mesh,
      scratch_types=[
          pltpu.VMEM((EPW,), jnp.int32),
          pltpu.VMEM((NP,), jnp.float32),
      ],
      compiler_params=pltpu.CompilerParams(needs_layout_passes=False),
  )(dstp)


def _agg_body(hp_hbm, srcm_hbm, dstm_hbm, zeros_hbm, out_hbm,
              src2_v, dst2_v, rows0_v, rows1_v, table_sh, acc_sh,
              sg0, sg1, ss0, ss1):
  cid = lax.axis_index("c")
  sid = lax.axis_index("s")
  w = cid * NS + sid
  r0 = sid * ROWS_PER_TILE
  pltpu.sync_copy(hp_hbm.at[pl.ds(r0, ROWS_PER_TILE)],
                  table_sh.at[pl.ds(r0, ROWS_PER_TILE)])
  pltpu.sync_copy(zeros_hbm.at[pl.ds(r0, ROWS_PER_TILE)],
                  acc_sh.at[pl.ds(r0, ROWS_PER_TILE)])
  pltpu.sync_copy(srcm_hbm.at[pl.ds(w * CH_PER_W, CH_PER_W)], src2_v)
  pltpu.sync_copy(dstm_hbm.at[pl.ds(w * CH_PER_W, CH_PER_W)], dst2_v)
  plsc.subcore_barrier()

  pltpu.async_copy(table_sh.at[src2_v.at[0]], rows0_v, sg0)

  @pl.loop(0, CH_PER_W, step=2)
  def _(k):
    pltpu.make_async_copy(table_sh.at[src2_v.at[k]], rows0_v, sg0).wait()

    @pl.when(k > 0)
    def _():
      pltpu.make_async_copy(rows1_v, acc_sh.at[dst2_v.at[k - 1]], ss1).wait()

    pltpu.async_copy(table_sh.at[src2_v.at[k + 1]], rows1_v, sg1)
    pltpu.async_copy(rows0_v, acc_sh.at[dst2_v.at[k]], ss0, add=True)
    pltpu.make_async_copy(table_sh.at[src2_v.at[k + 1]], rows1_v, sg1).wait()
    pltpu.make_async_copy(rows0_v, acc_sh.at[dst2_v.at[k]], ss0).wait()
    knext = jnp.where(k + 2 >= CH_PER_W, 0, k + 2)
    pltpu.async_copy(table_sh.at[src2_v.at[knext]], rows0_v, sg0)
    pltpu.async_copy(rows1_v, acc_sh.at[dst2_v.at[k + 1]], ss1, add=True)

  pltpu.make_async_copy(rows1_v, acc_sh.at[dst2_v.at[CH_PER_W - 1]],
                        ss1).wait()
  pltpu.make_async_copy(table_sh.at[src2_v.at[0]], rows0_v, sg0).wait()

  plsc.subcore_barrier()
  pltpu.sync_copy(acc_sh.at[pl.ds(r0, ROWS_PER_TILE)],
                  out_hbm.at[cid, pl.ds(r0, ROWS_PER_TILE)])


def _agg(hp, srcm, dstm, zeros_np, f):
  mesh = plsc.VectorSubcoreMesh(**_MESH)
  return pl.kernel(
      _agg_body,
      out_type=jax.ShapeDtypeStruct((NC, NP, f), jnp.float32),
      mesh=mesh,
      scratch_types=[
          pltpu.VMEM((CH_PER_W, CHUNK), jnp.int32),
          pltpu.VMEM((CH_PER_W, CHUNK), jnp.int32),
          pltpu.VMEM((CHUNK, f), jnp.float32),
          pltpu.VMEM((CHUNK, f), jnp.float32),
          pltpu.VMEM_SHARED((NP, f), jnp.float32),
          pltpu.VMEM_SHARED((NP, f), jnp.float32),
          pltpu.SemaphoreType.DMA,
          pltpu.SemaphoreType.DMA,
          pltpu.SemaphoreType.DMA,
          pltpu.SemaphoreType.DMA,
      ],
      compiler_params=pltpu.CompilerParams(use_tc_tiling_on_sc=False),
  )(hp, srcm, dstm, zeros_np)


def _mm1_body(x_ref, w1_ref, h1_ref):
  h1_ref[...] = jnp.dot(x_ref[...], w1_ref[...],
                        preferred_element_type=jnp.float32,
                        precision=lax.Precision.HIGHEST)


def _mm1(x, W1):
  bm = 1000
  return pl.pallas_call(
      _mm1_body,
      grid=(N // bm,),
      in_specs=[
          pl.BlockSpec((bm, D_IN), lambda i: (i, 0)),
          pl.BlockSpec((D_IN, NHID), lambda i: (0, 0)),
      ],
      out_specs=pl.BlockSpec((bm, NHID), lambda i: (i, 0)),
      out_shape=jax.ShapeDtypeStruct((NP, NHID), jnp.float32),
  )(x, W1)


def _dinv_body(degp_ref, h1_ref, dinv_ref, h1p_ref):
  deg = jnp.sum(degp_ref[...], axis=0) + 1.0
  dinv = lax.rsqrt(deg)[:, None]
  dinv_ref[...] = dinv
  h1p_ref[...] = h1_ref[...] * dinv


def _dinv(degp, h1):
  return pl.pallas_call(
      _dinv_body,
      grid=(1,),
      in_specs=[
          pl.BlockSpec((NW, NP), lambda i: (0, 0)),
          pl.BlockSpec((NP, NHID), lambda i: (0, 0)),
      ],
      out_specs=[
          pl.BlockSpec((NP, 1), lambda i: (0, 0)),
          pl.BlockSpec((NP, NHID), lambda i: (0, 0)),
      ],
      out_shape=[
          jax.ShapeDtypeStruct((NP, 1), jnp.float32),
          jax.ShapeDtypeStruct((NP, NHID), jnp.float32),
      ],
  )(degp, h1)


def _mid_body(p_ref, h1p_ref, dinv_ref, b1_ref, w2_ref, h2p_ref):
  dinv = dinv_ref[...]
  s = (p_ref[0] + p_ref[1] + h1p_ref[...]) * dinv + b1_ref[...]
  h = jnp.maximum(s, 0.0)
  h2 = jnp.dot(h, w2_ref[...], preferred_element_type=jnp.float32,
               precision=lax.Precision.HIGHEST)
  h2p_ref[...] = h2 * dinv


def _mid(p, h1p, dinv, b1, W2):
  bm = 1000
  return pl.pallas_call(
      _mid_body,
      grid=(N // bm,),
      in_specs=[
          pl.BlockSpec((NC, bm, NHID), lambda i: (0, i, 0)),
          pl.BlockSpec((bm, NHID), lambda i: (i, 0)),
          pl.BlockSpec((bm, 1), lambda i: (i, 0)),
          pl.BlockSpec((1, NHID), lambda i: (0, 0)),
          pl.BlockSpec((NHID, NLAT), lambda i: (0, 0)),
      ],
      out_specs=pl.BlockSpec((bm, NLAT), lambda i: (i, 0)),
      out_shape=jax.ShapeDtypeStruct((NP, NLAT), jnp.float32),
  )(p, h1p, dinv, b1, W2)


def _zk_body(q_ref, h2p_ref, dinv_ref, b2_ref, z_ref):
  z_ref[...] = ((q_ref[0] + q_ref[1] + h2p_ref[...]) * dinv_ref[...]
                + b2_ref[...])


def _zk(q, h2p, dinv, b2):
  bm = 1000
  return pl.pallas_call(
      _zk_body,
      grid=(N // bm,),
      in_specs=[
          pl.BlockSpec((NC, bm, NLAT), lambda i: (0, i, 0)),
          pl.BlockSpec((bm, NLAT), lambda i: (i, 0)),
          pl.BlockSpec((bm, 1), lambda i: (i, 0)),
          pl.BlockSpec((1, NLAT), lambda i: (0, 0)),
      ],
      out_specs=pl.BlockSpec((bm, NLAT), lambda i: (i, 0)),
      out_shape=jax.ShapeDtypeStruct((N, NLAT), jnp.float32),
  )(q, h2p, dinv, b2)


def _dec_body(zi_ref, zj_ref, out_ref):
  out_ref[...] = lax.dot_general(
      zi_ref[...], zj_ref[...],
      dimension_numbers=(((1,), (1,)), ((), ())),
      preferred_element_type=jnp.float32,
      precision=lax.Precision.HIGHEST)


def _dec(z):
  bm = 400
  return pl.pallas_call(
      _dec_body,
      grid=(N // bm,),
      in_specs=[
          pl.BlockSpec((bm, NLAT), lambda i: (i, 0)),
          pl.BlockSpec((N, NLAT), lambda i: (0, 0)),
      ],
      out_specs=pl.BlockSpec((bm, N), lambda i: (i, 0)),
      out_shape=jax.ShapeDtypeStruct((N, N), jnp.float32),
      compiler_params=pltpu.CompilerParams(
          dimension_semantics=("parallel",)),
  )(z, z)


def kernel(x, edge_index, W1, b1, W2, b2):
  src = edge_index[0]
  dst = edge_index[1]
  pad = EP - E
  srcp = jnp.concatenate([src, jnp.zeros((pad,), jnp.int32)])
  dstp = jnp.concatenate([dst, jnp.full((pad,), N, jnp.int32)])
  srcm = srcp.reshape(NW * CH_PER_W, CHUNK)
  dstm = dstp.reshape(NW * CH_PER_W, CHUNK)
  zeros64 = jnp.zeros((NP, NHID), jnp.float32)
  zeros16 = jnp.zeros((NP, NLAT), jnp.float32)

  degp = _deg(dstp)
  h1 = _mm1(x, W1)
  dinv, h1p = _dinv(degp, h1)
  p = _agg(h1p, srcm, dstm, zeros64, NHID)
  h2p = _mid(p, h1p, dinv, b1.reshape(1, NHID), W2)
  q = _agg(h2p, srcm, dstm, zeros16, NLAT)
  z = _zk(q, h2p, dinv, b2.reshape(1, NLAT))
  return _dec(z)


# speedup vs baseline: 28.7658x; 1.5804x over previous
import functools

import jax
import jax.numpy as jnp
from jax import lax
from jax.experimental import pallas as pl
from jax.experimental.pallas import tpu as pltpu
from jax.experimental.pallas import tpu_sc as plsc

N = 10000
E = 320000
D_IN = 128
NHID = 64
NLAT = 16

NC = 2
NS = 16
L = 16
NW = NC * NS
CHUNK = 128
CH_PER_W = 80
EPW = CH_PER_W * CHUNK
EP = NW * EPW
NP = 10112
ROWS_PER_TILE = NP // NS

_MESH = dict(core_axis_name="c", subcore_axis_name="s")


def _deg_body(dst_hbm, out_hbm, idx_v, deg_v):
  cid = lax.axis_index("c")
  sid = lax.axis_index("s")
  w = cid * NS + sid
  pltpu.sync_copy(dst_hbm.at[pl.ds(w * EPW, EPW)], idx_v)

  @pl.loop(0, NP // L)
  def _(i):
    deg_v[pl.ds(i * L, L)] = jnp.zeros((L,), jnp.float32)

  ones = jnp.ones((L,), jnp.float32)

  @pl.loop(0, EPW // L)
  def _(j):
    idx = idx_v[pl.ds(j * L, L)]
    plsc.addupdate_scatter(deg_v, [idx], ones)

  pltpu.sync_copy(deg_v, out_hbm.at[w])


def _deg(dstp):
  mesh = plsc.VectorSubcoreMesh(**_MESH)
  return pl.kernel(
      _deg_body,
      out_type=jax.ShapeDtypeStruct((NW, NP), jnp.float32),
      mesh=mesh,
      scratch_types=[
          pltpu.VMEM((EPW,), jnp.int32),
          pltpu.VMEM((NP,), jnp.float32),
      ],
      compiler_params=pltpu.CompilerParams(needs_layout_passes=False),
  )(dstp)


def _agg_body(hp_hbm, srcm_hbm, dstm_hbm, zeros_hbm, out_hbm,
              src2_v, dst2_v, rows0_v, rows1_v, table_sh, acc_sh,
              sg0, sg1, ss0, ss1):
  cid = lax.axis_index("c")
  sid = lax.axis_index("s")
  w = cid * NS + sid
  r0 = sid * ROWS_PER_TILE
  pltpu.sync_copy(hp_hbm.at[pl.ds(r0, ROWS_PER_TILE)],
                  table_sh.at[pl.ds(r0, ROWS_PER_TILE)])
  pltpu.sync_copy(zeros_hbm.at[pl.ds(r0, ROWS_PER_TILE)],
                  acc_sh.at[pl.ds(r0, ROWS_PER_TILE)])
  pltpu.sync_copy(srcm_hbm.at[pl.ds(w * CH_PER_W, CH_PER_W)], src2_v)
  pltpu.sync_copy(dstm_hbm.at[pl.ds(w * CH_PER_W, CH_PER_W)], dst2_v)
  plsc.subcore_barrier()

  pltpu.async_copy(table_sh.at[src2_v.at[0]], rows0_v, sg0)

  @pl.loop(0, CH_PER_W, step=2)
  def _(k):
    pltpu.make_async_copy(table_sh.at[src2_v.at[k]], rows0_v, sg0).wait()

    @pl.when(k > 0)
    def _():
      pltpu.make_async_copy(rows1_v, acc_sh.at[dst2_v.at[k - 1]], ss1).wait()

    pltpu.async_copy(table_sh.at[src2_v.at[k + 1]], rows1_v, sg1)
    pltpu.async_copy(rows0_v, acc_sh.at[dst2_v.at[k]], ss0, add=True)
    pltpu.make_async_copy(table_sh.at[src2_v.at[k + 1]], rows1_v, sg1).wait()
    pltpu.make_async_copy(rows0_v, acc_sh.at[dst2_v.at[k]], ss0).wait()
    knext = jnp.where(k + 2 >= CH_PER_W, 0, k + 2)
    pltpu.async_copy(table_sh.at[src2_v.at[knext]], rows0_v, sg0)
    pltpu.async_copy(rows1_v, acc_sh.at[dst2_v.at[k + 1]], ss1, add=True)

  pltpu.make_async_copy(rows1_v, acc_sh.at[dst2_v.at[CH_PER_W - 1]],
                        ss1).wait()
  pltpu.make_async_copy(table_sh.at[src2_v.at[0]], rows0_v, sg0).wait()

  plsc.subcore_barrier()
  pltpu.sync_copy(acc_sh.at[pl.ds(r0, ROWS_PER_TILE)],
                  out_hbm.at[cid, pl.ds(r0, ROWS_PER_TILE)])


def _agg(hp, srcm, dstm, zeros_np, f):
  mesh = plsc.VectorSubcoreMesh(**_MESH)
  return pl.kernel(
      _agg_body,
      out_type=jax.ShapeDtypeStruct((NC, NP, f), jnp.float32),
      mesh=mesh,
      scratch_types=[
          pltpu.VMEM((CH_PER_W, CHUNK), jnp.int32),
          pltpu.VMEM((CH_PER_W, CHUNK), jnp.int32),
          pltpu.VMEM((CHUNK, f), jnp.float32),
          pltpu.VMEM((CHUNK, f), jnp.float32),
          pltpu.VMEM_SHARED((NP, f), jnp.float32),
          pltpu.VMEM_SHARED((NP, f), jnp.float32),
          pltpu.SemaphoreType.DMA,
          pltpu.SemaphoreType.DMA,
          pltpu.SemaphoreType.DMA,
          pltpu.SemaphoreType.DMA,
      ],
      compiler_params=pltpu.CompilerParams(use_tc_tiling_on_sc=False),
  )(hp, srcm, dstm, zeros_np)


def _mm1_body(x_ref, w1_ref, h1_ref):
  h1_ref[...] = jnp.dot(x_ref[...], w1_ref[...],
                        preferred_element_type=jnp.float32,
                        precision=lax.Precision.HIGHEST)


def _mm1(x, W1):
  bm = 1000
  return pl.pallas_call(
      _mm1_body,
      grid=(N // bm,),
      in_specs=[
          pl.BlockSpec((bm, D_IN), lambda i: (i, 0)),
          pl.BlockSpec((D_IN, NHID), lambda i: (0, 0)),
      ],
      out_specs=pl.BlockSpec((bm, NHID), lambda i: (i, 0)),
      out_shape=jax.ShapeDtypeStruct((NP, NHID), jnp.float32),
  )(x, W1)


def _dinv_body(degp_ref, h1_ref, dinv_ref, h1p_ref):
  deg = jnp.sum(degp_ref[...], axis=0) + 1.0
  dinv = lax.rsqrt(deg)[:, None]
  dinv_ref[...] = dinv
  h1p_ref[...] = h1_ref[...] * dinv


def _dinv(degp, h1):
  return pl.pallas_call(
      _dinv_body,
      grid=(1,),
      in_specs=[
          pl.BlockSpec((NW, NP), lambda i: (0, 0)),
          pl.BlockSpec((NP, NHID), lambda i: (0, 0)),
      ],
      out_specs=[
          pl.BlockSpec((NP, 1), lambda i: (0, 0)),
          pl.BlockSpec((NP, NHID), lambda i: (0, 0)),
      ],
      out_shape=[
          jax.ShapeDtypeStruct((NP, 1), jnp.float32),
          jax.ShapeDtypeStruct((NP, NHID), jnp.float32),
      ],
  )(degp, h1)


def _mid_body(p_ref, h1p_ref, dinv_ref, b1_ref, w2_ref, h2p_ref):
  dinv = dinv_ref[...]
  s = (p_ref[0] + p_ref[1] + h1p_ref[...]) * dinv + b1_ref[...]
  h = jnp.maximum(s, 0.0)
  h2 = jnp.dot(h, w2_ref[...], preferred_element_type=jnp.float32,
               precision=lax.Precision.HIGHEST)
  h2p_ref[...] = h2 * dinv


def _mid(p, h1p, dinv, b1, W2):
  bm = 1000
  return pl.pallas_call(
      _mid_body,
      grid=(N // bm,),
      in_specs=[
          pl.BlockSpec((NC, bm, NHID), lambda i: (0, i, 0)),
          pl.BlockSpec((bm, NHID), lambda i: (i, 0)),
          pl.BlockSpec((bm, 1), lambda i: (i, 0)),
          pl.BlockSpec((1, NHID), lambda i: (0, 0)),
          pl.BlockSpec((NHID, NLAT), lambda i: (0, 0)),
      ],
      out_specs=pl.BlockSpec((bm, NLAT), lambda i: (i, 0)),
      out_shape=jax.ShapeDtypeStruct((NP, NLAT), jnp.float32),
  )(p, h1p, dinv, b1, W2)


def _zk_body(q_ref, h2p_ref, dinv_ref, b2_ref, z_ref):
  z = (q_ref[0] + q_ref[1] + h2p_ref[...]) * dinv_ref[...] + b2_ref[...]
  z_ref[...] = z.astype(jnp.bfloat16)


def _zk(q, h2p, dinv, b2):
  bm = 1000
  return pl.pallas_call(
      _zk_body,
      grid=(N // bm,),
      in_specs=[
          pl.BlockSpec((NC, bm, NLAT), lambda i: (0, i, 0)),
          pl.BlockSpec((bm, NLAT), lambda i: (i, 0)),
          pl.BlockSpec((bm, 1), lambda i: (i, 0)),
          pl.BlockSpec((1, NLAT), lambda i: (0, 0)),
      ],
      out_specs=pl.BlockSpec((bm, NLAT), lambda i: (i, 0)),
      out_shape=jax.ShapeDtypeStruct((N, NLAT), jnp.bfloat16),
  )(q, h2p, dinv, b2)


def _dec_body(zi_ref, zj_ref, out_ref):
  out_ref[...] = lax.dot_general(
      zi_ref[...], zj_ref[...],
      dimension_numbers=(((1,), (1,)), ((), ())),
      preferred_element_type=jnp.float32)


def _dec(z):
  bm = 400
  return pl.pallas_call(
      _dec_body,
      grid=(N // bm,),
      in_specs=[
          pl.BlockSpec((bm, NLAT), lambda i: (i, 0)),
          pl.BlockSpec((N, NLAT), lambda i: (0, 0)),
      ],
      out_specs=pl.BlockSpec((bm, N), lambda i: (i, 0)),
      out_shape=jax.ShapeDtypeStruct((N, N), jnp.float32),
      compiler_params=pltpu.CompilerParams(
          dimension_semantics=("parallel",)),
  )(z, z)


def kernel(x, edge_index, W1, b1, W2, b2):
  src = edge_index[0]
  dst = edge_index[1]
  pad = EP - E
  srcp = jnp.concatenate([src, jnp.zeros((pad,), jnp.int32)])
  dstp = jnp.concatenate([dst, jnp.full((pad,), N, jnp.int32)])
  srcm = srcp.reshape(NW * CH_PER_W, CHUNK)
  dstm = dstp.reshape(NW * CH_PER_W, CHUNK)
  zeros64 = jnp.zeros((NP, NHID), jnp.float32)
  zeros16 = jnp.zeros((NP, NLAT), jnp.float32)

  degp = _deg(dstp)
  h1 = _mm1(x, W1)
  dinv, h1p = _dinv(degp, h1)
  p = _agg(h1p, srcm, dstm, zeros64, NHID)
  h2p = _mid(p, h1p, dinv, b1.reshape(1, NHID), W2)
  q = _agg(h2p, srcm, dstm, zeros16, NLAT)
  z = _zk(q, h2p, dinv, b2.reshape(1, NLAT))
  return _dec(z)
